# baseline (speedup 1.0000x reference)
import dataclasses

import jax
import jax.numpy as jnp
from jax import lax
from jax.experimental import pallas as pl
from jax.experimental.pallas import tpu as pltpu
from jax.experimental.pallas import tpu_sc as plsc

N, D, M, R, K = 4096, 1024, 32768, 64, 32
BN = 128
BM = 16384
BLK = 128
NB = M // BLK
CH = 256
NI, NJ = CH // BN, M // BM
BPJ = BM // BLK

NW = 32
NCH = N // CH
ROWS = CH // NW
NEG = -3.0e38
BIGI = 1 << 30
CAP = 256
TRIG = 96


def _enc_body(x_ref, b_full_ref, a_ref, bias_ref, bdec_ref,
              pre_ref, bmax_ref, t0_ref, inter_s, bmax_s):
    j = pl.program_id(1)

    @pl.when(j == 0)
    def _():
        xc = x_ref[...] - bdec_ref[...]
        inter_s[...] = lax.dot_general(
            xc, b_full_ref[...], (((1,), (1,)), ((), ())),
            preferred_element_type=jnp.float32)

    pre = lax.dot_general(
        inter_s[...], a_ref[...], (((1,), (1,)), ((), ())),
        preferred_element_type=jnp.float32) + bias_ref[...]
    pre_ref[...] = pre
    bm = jnp.max(pre.reshape(BN, BPJ, BLK), axis=2)
    bmax_s[:, pl.ds(j * BPJ, BPJ)] = bm

    @pl.when(j == NJ - 1)
    def _():
        bmax_ref[...] = bmax_s[...]
        sel = bmax_s[...]
        iota = lax.broadcasted_iota(jnp.int32, (BN, NB), 1)
        m = jnp.max(sel, axis=1, keepdims=True)
        for _ in range(K):
            m = jnp.max(sel, axis=1, keepdims=True)
            am = jnp.min(jnp.where(sel == m, iota, BIGI), axis=1, keepdims=True)
            sel = jnp.where(iota == am, NEG, sel)
        t0_ref[...] = m


def _encode(x, A, B, bias, b_dec):
    bias2 = bias.reshape(1, M)
    bdec2 = b_dec.reshape(1, D)
    grid = (NI, NJ)
    return pl.pallas_call(
        _enc_body,
        grid=grid,
        in_specs=[
            pl.BlockSpec((BN, D), lambda i, j: (i, 0)),
            pl.BlockSpec((R, D), lambda i, j: (0, 0)),
            pl.BlockSpec((BM, R), lambda i, j: (j, 0)),
            pl.BlockSpec((1, BM), lambda i, j: (0, j)),
            pl.BlockSpec((1, D), lambda i, j: (0, 0)),
        ],
        out_specs=[
            pl.BlockSpec((BN, BM), lambda i, j: (i, j)),
            pl.BlockSpec((BN, NB), lambda i, j: (i, 0)),
            pl.BlockSpec((BN, 1), lambda i, j: (i, 0)),
        ],
        out_shape=[
            jax.ShapeDtypeStruct((CH, M), jnp.float32),
            jax.ShapeDtypeStruct((CH, NB), jnp.float32),
            jax.ShapeDtypeStruct((CH, 1), jnp.float32),
        ],
        scratch_shapes=[
            pltpu.VMEM((BN, R), jnp.float32),
            pltpu.VMEM((BN, NB), jnp.float32),
        ],
        compiler_params=pltpu.CompilerParams(
            dimension_semantics=("arbitrary", "arbitrary")),
    )(x, B, A, bias2, bdec2)


def _iota16():
    return lax.iota(jnp.int32, 16)


def _splat(v, dtype=jnp.int32):
    return jnp.full((16,), v, dtype)


def _scalar_max(v):
    return jnp.max(v)


def _sc_body(preb, bmaxh, t0h, wdech, bdech, outh,
             bmax_c, t0_v, bdec_v, candg_v, candl_v, blkbuf,
             val_b, idx_b, tmp_b, wbuf, acc_v, sem):
    wid = lax.axis_index("s") * 2 + lax.axis_index("c")
    base = wid * ROWS
    it16 = _iota16()

    pltpu.sync_copy(bdech, bdec_v)
    pltpu.sync_copy(t0h.at[pl.ds(base, ROWS)], t0_v)
    pltpu.sync_copy(bmaxh.at[pl.ds(base, ROWS)], bmax_c)

    def compress(nc):
        nch = (nc + 15) // 16
        val_b[pl.ds(nc, 16)] = _splat(NEG, jnp.float32)
        idx_b[pl.ds(nc, 16)] = _splat(BIGI, jnp.int32)

        def copy_chunk(q, _):
            tmp_b[pl.ds(q * 16, 16)] = val_b[pl.ds(q * 16, 16)]
            return 0
        lax.fori_loop(0, nch, copy_chunk, 0)

        def sel_step(_, carry):
            def maxv(q, acc):
                return jnp.maximum(acc, jnp.max(tmp_b[pl.ds(q * 16, 16)]))
            m = lax.fori_loop(0, nch, maxv, jnp.float32(NEG))

            def minix(q, acc):
                v = tmp_b[pl.ds(q * 16, 16)]
                ix = idx_b[pl.ds(q * 16, 16)]
                return jnp.minimum(
                    acc, jnp.min(jnp.where(v == m, ix, BIGI)))
            mi = lax.fori_loop(0, nch, minix, jnp.int32(BIGI))

            def knock(q, _):
                v = tmp_b[pl.ds(q * 16, 16)]
                ix = idx_b[pl.ds(q * 16, 16)]
                tmp_b[pl.ds(q * 16, 16)] = jnp.where(
                    (v == m) & (ix == mi), NEG, v)
                return 0
            lax.fori_loop(0, nch, knock, 0)
            return (m, mi)

        t32, i32s = lax.fori_loop(
            0, K, sel_step, (jnp.float32(NEG), jnp.int32(BIGI)))

        def compact(q, cur):
            v = val_b[pl.ds(q * 16, 16)]
            ix = idx_b[pl.ds(q * 16, 16)]
            keep = (v > t32) | ((v == t32) & (ix <= i32s))
            pos = cur + plsc.cumsum(keep.astype(jnp.int32)) - 1
            plsc.store_scatter(val_b, [pos], v, mask=keep)
            plsc.store_scatter(idx_b, [pos], ix, mask=keep)
            return cur + _scalar_max(
                plsc.all_reduce_population_count(keep))
        lax.fori_loop(0, nch, compact, jnp.int32(0))
        return jnp.int32(K)

    def row_body(r, _):
        g = base + r
        t0s = plsc.load_gather(t0_v, [_splat(r)])

        curv = _splat(0)
        padv = _splat(BIGI)
        for jb in range(NB // 16):
            bm = bmax_c[r, pl.ds(jb * 16, 16)]
            msk = bm >= t0s
            ids = jb * 16 + it16
            pos = curv + plsc.cumsum(msk.astype(jnp.int32)) - 1
            plsc.store_scatter(candl_v, [pos], ids, mask=msk)
            plsc.store_scatter(candg_v, [pos], g * NB + ids, mask=msk)
            curv = curv + plsc.all_reduce_population_count(msk)
            padv = jnp.minimum(padv, jnp.where(msk, BIGI, ids))
        nbk = curv[0]
        padid = jnp.minimum(jnp.min(padv), NB - 1)
        candl_v[pl.ds(nbk, 16)] = _splat(padid)
        candg_v[pl.ds(nbk, 16)] = _splat(g * NB + padid)
        ngrp = (nbk + 7) // 8

        def fire(q, _):
            pltpu.async_copy(
                preb.at[candg_v.at[pl.ds(q * 8, 8)]],
                blkbuf.at[pl.ds(q * 8, 8)], sem)
            return 0
        lax.fori_loop(0, ngrp, fire, 0)

        def drain(q, _):
            pltpu.make_async_copy(
                preb.at[candg_v.at[pl.ds(0, 8)]],
                blkbuf.at[pl.ds(0, 8)], sem).wait()
            return 0
        lax.fori_loop(0, ngrp, drain, 0)

        def maybe_compress(curv2, trig):
            nc_s = curv2[0]

            def do(_):
                compress(nc_s)
                return _splat(K)
            return lax.cond(nc_s > trig, do, lambda _: curv2, 0)

        def scan_block(b, curv2):
            lidv = plsc.load_gather(candl_v, [_splat(b)])
            for c in range(BLK // 16):
                v = blkbuf[b, pl.ds(c * 16, 16)]
                msk = v >= t0s
                gix = lidv * BLK + (c * 16 + it16)
                pos = curv2 + plsc.cumsum(msk.astype(jnp.int32)) - 1
                plsc.store_scatter(val_b, [pos], v, mask=msk)
                plsc.store_scatter(idx_b, [pos], gix, mask=msk)
                curv2 = curv2 + plsc.all_reduce_population_count(msk)
            return maybe_compress(curv2, TRIG)
        ncv = lax.fori_loop(0, ngrp * 8, scan_block, _splat(0))
        maybe_compress(ncv, K)

        pltpu.async_copy(
            wdech.at[idx_b.at[pl.ds(0, K)]], wbuf, sem).wait()
        half = D // 2
        for h in range(2):
            off = h * half

            def dec_k(k, accs):
                vk = jnp.maximum(
                    plsc.load_gather(val_b, [_splat(k)]), jnp.float32(0.0))
                return tuple(
                    accs[c] + wbuf[k, pl.ds(off + c * 16, 16)] * vk
                    for c in range(half // 16))
            init = tuple(
                bdec_v[pl.ds(off + c * 16, 16)] for c in range(half // 16))
            accs = lax.fori_loop(0, K, dec_k, init)
            for c in range(half // 16):
                acc_v[pl.ds(off + c * 16, 16)] = accs[c]

        pltpu.sync_copy(acc_v, outh.at[g])
        return 0

    lax.fori_loop(0, ROWS, row_body, 0)


def _select_decode(preb, bmax, t0, W_dec, b_dec):
    mesh = plsc.VectorSubcoreMesh(core_axis_name="c", subcore_axis_name="s")
    cp = pltpu.CompilerParams()
    if "needs_layout_passes" in pltpu.CompilerParams.__dataclass_fields__:
        cp = dataclasses.replace(cp, needs_layout_passes=False)
    kern = pl.kernel(
        _sc_body,
        out_type=jax.ShapeDtypeStruct((CH, D), jnp.float32),
        mesh=mesh,
        compiler_params=cp,
        scratch_types=[
            pltpu.VMEM((ROWS, NB), jnp.float32),
            pltpu.VMEM((ROWS,), jnp.float32),
            pltpu.VMEM((D,), jnp.float32),
            pltpu.VMEM((NB + 16,), jnp.int32),
            pltpu.VMEM((NB + 16,), jnp.int32),
            pltpu.VMEM((NB + 8, BLK), jnp.float32),
            pltpu.VMEM((CAP + 16,), jnp.float32),
            pltpu.VMEM((CAP + 16,), jnp.int32),
            pltpu.VMEM((CAP + 16,), jnp.float32),
            pltpu.VMEM((K, D), jnp.float32),
            pltpu.VMEM((D,), jnp.float32),
            pltpu.SemaphoreType.DMA,
        ],
    )
    return kern(preb, bmax, t0, W_dec, b_dec)


def kernel(x, A, B, bias, W_dec, b_dec):
    outs = []
    for ch in range(NCH):
        xs = lax.slice_in_dim(x, ch * CH, (ch + 1) * CH, axis=0)
        pre, bmax, t0 = _encode(xs, A, B, bias, b_dec)
        preb = pre.reshape(CH * NB, BLK)
        outs.append(_select_decode(preb, bmax, t0.reshape(CH), W_dec, b_dec))
    return jnp.concatenate(outs, axis=0)


# speedup vs baseline: 1.0193x; 1.0193x over previous
import dataclasses

import jax
import jax.numpy as jnp
from jax import lax
from jax.experimental import pallas as pl
from jax.experimental.pallas import tpu as pltpu
from jax.experimental.pallas import tpu_sc as plsc

N, D, M, R, K = 4096, 1024, 32768, 64, 32
BN = 128
BM = 16384
BLK = 128
NB = M // BLK
CH = 512
NI, NJ = CH // BN, M // BM
BPJ = BM // BLK

NW = 32
NCH = N // CH
ROWS = CH // NW
NEG = -3.0e38
BIGI = 1 << 30
CAP = 256
TRIG = 96


def _enc_body(x_ref, b_full_ref, a_ref, bias_ref, bdec_ref,
              pre_ref, bmax_ref, t0_ref, inter_s, bmax_s):
    j = pl.program_id(1)

    @pl.when(j == 0)
    def _():
        xc = x_ref[...] - bdec_ref[...]
        inter_s[...] = lax.dot_general(
            xc, b_full_ref[...], (((1,), (1,)), ((), ())),
            preferred_element_type=jnp.float32)

    pre = lax.dot_general(
        inter_s[...], a_ref[...], (((1,), (1,)), ((), ())),
        preferred_element_type=jnp.float32) + bias_ref[...]
    pre_ref[...] = pre
    bm = jnp.max(pre.reshape(BN, BPJ, BLK), axis=2)
    bmax_s[:, pl.ds(j * BPJ, BPJ)] = bm

    @pl.when(j == NJ - 1)
    def _():
        bmax_ref[...] = bmax_s[...]
        sel = bmax_s[...]
        iota = lax.broadcasted_iota(jnp.int32, (BN, NB), 1)
        m = jnp.max(sel, axis=1, keepdims=True)
        for _ in range(K):
            m = jnp.max(sel, axis=1, keepdims=True)
            am = jnp.min(jnp.where(sel == m, iota, BIGI), axis=1, keepdims=True)
            sel = jnp.where(iota == am, NEG, sel)
        t0_ref[...] = m


def _encode(x, A, B, bias, b_dec):
    bias2 = bias.reshape(1, M)
    bdec2 = b_dec.reshape(1, D)
    grid = (NI, NJ)
    return pl.pallas_call(
        _enc_body,
        grid=grid,
        in_specs=[
            pl.BlockSpec((BN, D), lambda i, j: (i, 0)),
            pl.BlockSpec((R, D), lambda i, j: (0, 0)),
            pl.BlockSpec((BM, R), lambda i, j: (j, 0)),
            pl.BlockSpec((1, BM), lambda i, j: (0, j)),
            pl.BlockSpec((1, D), lambda i, j: (0, 0)),
        ],
        out_specs=[
            pl.BlockSpec((BN, BM), lambda i, j: (i, j)),
            pl.BlockSpec((BN, NB), lambda i, j: (i, 0)),
            pl.BlockSpec((BN, 1), lambda i, j: (i, 0)),
        ],
        out_shape=[
            jax.ShapeDtypeStruct((CH, M), jnp.float32),
            jax.ShapeDtypeStruct((CH, NB), jnp.float32),
            jax.ShapeDtypeStruct((CH, 1), jnp.float32),
        ],
        scratch_shapes=[
            pltpu.VMEM((BN, R), jnp.float32),
            pltpu.VMEM((BN, NB), jnp.float32),
        ],
        compiler_params=pltpu.CompilerParams(
            dimension_semantics=("arbitrary", "arbitrary")),
    )(x, B, A, bias2, bdec2)


def _iota16():
    return lax.iota(jnp.int32, 16)


def _splat(v, dtype=jnp.int32):
    return jnp.full((16,), v, dtype)


def _scalar_max(v):
    return jnp.max(v)


def _sc_body(preb, bmaxh, t0h, wdech, bdech, outh,
             bmax_c, t0_v, bdec_v, candg_v, candl_v, blkbuf,
             val_b, idx_b, tmp_b, wbuf, acc_v, sem):
    wid = lax.axis_index("s") * 2 + lax.axis_index("c")
    base = wid * ROWS
    it16 = _iota16()

    pltpu.sync_copy(bdech, bdec_v)
    pltpu.sync_copy(t0h.at[pl.ds(base, ROWS)], t0_v)
    pltpu.sync_copy(bmaxh.at[pl.ds(base, ROWS)], bmax_c)

    def compress(nc):
        nch = (nc + 15) // 16
        val_b[pl.ds(nc, 16)] = _splat(NEG, jnp.float32)
        idx_b[pl.ds(nc, 16)] = _splat(BIGI, jnp.int32)

        def copy_chunk(q, _):
            tmp_b[pl.ds(q * 16, 16)] = val_b[pl.ds(q * 16, 16)]
            return 0
        lax.fori_loop(0, nch, copy_chunk, 0)

        def sel_step(_, carry):
            def maxv(q, acc):
                return jnp.maximum(acc, jnp.max(tmp_b[pl.ds(q * 16, 16)]))
            m = lax.fori_loop(0, nch, maxv, jnp.float32(NEG))

            def minix(q, acc):
                v = tmp_b[pl.ds(q * 16, 16)]
                ix = idx_b[pl.ds(q * 16, 16)]
                return jnp.minimum(
                    acc, jnp.min(jnp.where(v == m, ix, BIGI)))
            mi = lax.fori_loop(0, nch, minix, jnp.int32(BIGI))

            def knock(q, _):
                v = tmp_b[pl.ds(q * 16, 16)]
                ix = idx_b[pl.ds(q * 16, 16)]
                tmp_b[pl.ds(q * 16, 16)] = jnp.where(
                    (v == m) & (ix == mi), NEG, v)
                return 0
            lax.fori_loop(0, nch, knock, 0)
            return (m, mi)

        t32, i32s = lax.fori_loop(
            0, K, sel_step, (jnp.float32(NEG), jnp.int32(BIGI)))

        def compact(q, cur):
            v = val_b[pl.ds(q * 16, 16)]
            ix = idx_b[pl.ds(q * 16, 16)]
            keep = (v > t32) | ((v == t32) & (ix <= i32s))
            pos = cur + plsc.cumsum(keep.astype(jnp.int32)) - 1
            plsc.store_scatter(val_b, [pos], v, mask=keep)
            plsc.store_scatter(idx_b, [pos], ix, mask=keep)
            return cur + _scalar_max(
                plsc.all_reduce_population_count(keep))
        lax.fori_loop(0, nch, compact, jnp.int32(0))
        return jnp.int32(K)

    def row_body(r, _):
        g = base + r
        t0s = plsc.load_gather(t0_v, [_splat(r)])

        curv = _splat(0)
        padv = _splat(BIGI)
        for jb in range(NB // 16):
            bm = bmax_c[r, pl.ds(jb * 16, 16)]
            msk = bm >= t0s
            ids = jb * 16 + it16
            pos = curv + plsc.cumsum(msk.astype(jnp.int32)) - 1
            plsc.store_scatter(candl_v, [pos], ids, mask=msk)
            plsc.store_scatter(candg_v, [pos], g * NB + ids, mask=msk)
            curv = curv + plsc.all_reduce_population_count(msk)
            padv = jnp.minimum(padv, jnp.where(msk, BIGI, ids))
        nbk = curv[0]
        padid = jnp.minimum(jnp.min(padv), NB - 1)
        candl_v[pl.ds(nbk, 16)] = _splat(padid)
        candg_v[pl.ds(nbk, 16)] = _splat(g * NB + padid)
        ngrp = (nbk + 7) // 8

        def fire(q, _):
            pltpu.async_copy(
                preb.at[candg_v.at[pl.ds(q * 8, 8)]],
                blkbuf.at[pl.ds(q * 8, 8)], sem)
            return 0
        lax.fori_loop(0, ngrp, fire, 0)

        def drain(q, _):
            pltpu.make_async_copy(
                preb.at[candg_v.at[pl.ds(0, 8)]],
                blkbuf.at[pl.ds(0, 8)], sem).wait()
            return 0
        lax.fori_loop(0, ngrp, drain, 0)

        def maybe_compress(curv2, trig):
            nc_s = curv2[0]

            def do(_):
                compress(nc_s)
                return _splat(K)
            return lax.cond(nc_s > trig, do, lambda _: curv2, 0)

        def scan_block(b, curv2):
            lidv = plsc.load_gather(candl_v, [_splat(b)])
            for c in range(BLK // 16):
                v = blkbuf[b, pl.ds(c * 16, 16)]
                msk = v >= t0s
                gix = lidv * BLK + (c * 16 + it16)
                pos = curv2 + plsc.cumsum(msk.astype(jnp.int32)) - 1
                plsc.store_scatter(val_b, [pos], v, mask=msk)
                plsc.store_scatter(idx_b, [pos], gix, mask=msk)
                curv2 = curv2 + plsc.all_reduce_population_count(msk)
            return maybe_compress(curv2, TRIG)
        ncv = lax.fori_loop(0, ngrp * 8, scan_block, _splat(0))
        maybe_compress(ncv, K)

        pltpu.async_copy(
            wdech.at[idx_b.at[pl.ds(0, K)]], wbuf, sem).wait()
        half = D // 2
        for h in range(2):
            off = h * half

            def dec_k(k, accs):
                vk = jnp.maximum(
                    plsc.load_gather(val_b, [_splat(k)]), jnp.float32(0.0))
                return tuple(
                    accs[c] + wbuf[k, pl.ds(off + c * 16, 16)] * vk
                    for c in range(half // 16))
            init = tuple(
                bdec_v[pl.ds(off + c * 16, 16)] for c in range(half // 16))
            accs = lax.fori_loop(0, K, dec_k, init)
            for c in range(half // 16):
                acc_v[pl.ds(off + c * 16, 16)] = accs[c]

        pltpu.sync_copy(acc_v, outh.at[g])
        return 0

    lax.fori_loop(0, ROWS, row_body, 0)


def _select_decode(preb, bmax, t0, W_dec, b_dec):
    mesh = plsc.VectorSubcoreMesh(core_axis_name="c", subcore_axis_name="s")
    cp = pltpu.CompilerParams()
    if "needs_layout_passes" in pltpu.CompilerParams.__dataclass_fields__:
        cp = dataclasses.replace(cp, needs_layout_passes=False)
    kern = pl.kernel(
        _sc_body,
        out_type=jax.ShapeDtypeStruct((CH, D), jnp.float32),
        mesh=mesh,
        compiler_params=cp,
        scratch_types=[
            pltpu.VMEM((ROWS, NB), jnp.float32),
            pltpu.VMEM((ROWS,), jnp.float32),
            pltpu.VMEM((D,), jnp.float32),
            pltpu.VMEM((NB + 16,), jnp.int32),
            pltpu.VMEM((NB + 16,), jnp.int32),
            pltpu.VMEM((NB + 8, BLK), jnp.float32),
            pltpu.VMEM((CAP + 16,), jnp.float32),
            pltpu.VMEM((CAP + 16,), jnp.int32),
            pltpu.VMEM((CAP + 16,), jnp.float32),
            pltpu.VMEM((K, D), jnp.float32),
            pltpu.VMEM((D,), jnp.float32),
            pltpu.SemaphoreType.DMA,
        ],
    )
    return kern(preb, bmax, t0, W_dec, b_dec)


def kernel(x, A, B, bias, W_dec, b_dec):
    outs = []
    for ch in range(NCH):
        xs = lax.slice_in_dim(x, ch * CH, (ch + 1) * CH, axis=0)
        pre, bmax, t0 = _encode(xs, A, B, bias, b_dec)
        preb = pre.reshape(CH * NB, BLK)
        outs.append(_select_decode(preb, bmax, t0.reshape(CH), W_dec, b_dec))
    return jnp.concatenate(outs, axis=0)
